# baseline (speedup 1.0000x reference)
import functools
import jax
import jax.numpy as jnp
from jax import lax
from jax.experimental import pallas as pl
from jax.experimental.pallas import tpu as pltpu
from jax.experimental.pallas import tpu_sc as plsc

NE = 8
D = 1024
H = 2048
S = 2048
LANES = 128
T = 256
NB = 24
NBT = NB * T
TS = 512

NC = 2
NS = 16
NW = NC * NS
RPW = NBT // NW
GCH = 48
TPW = S // NW
CCH = 16


def _route_body(x_ref, rw_ref, rb_ref, logits_ref, dest_ref, w_ref,
                bexp_ref):
    x = x_ref[...]
    logits = lax.dot_general(
        x, rw_ref[...], (((1,), (0,)), ((), ())),
        preferred_element_type=jnp.float32) + rb_ref[...]
    lane = lax.broadcasted_iota(jnp.int32, (S, LANES), 1)
    valid = lane < NE
    lmask = jnp.where(valid, logits, -jnp.inf)
    mx = jnp.max(lmask, axis=-1, keepdims=True)
    p = jnp.exp(lmask - mx)
    probs = p / jnp.sum(p, axis=-1, keepdims=True)
    m1 = jnp.max(probs, axis=-1, keepdims=True)
    i1 = jnp.min(jnp.where((probs == m1) & valid, lane, LANES), axis=-1,
                 keepdims=True)
    sel1 = lane == i1
    probs2 = jnp.where(sel1, -1.0, probs)
    m2 = jnp.max(probs2, axis=-1, keepdims=True)
    i2 = jnp.min(jnp.where((probs2 == m2) & valid, lane, LANES), axis=-1,
                 keepdims=True)
    wsum = m1 + m2
    w0 = m1 / wsum
    w1 = m2 / wsum

    onehot0 = jnp.where(sel1, 1.0, 0.0)
    onehot1 = jnp.where(lane == i2, 1.0, 0.0)

    ri = lax.broadcasted_iota(jnp.int32, (TS, TS), 0)
    ci = lax.broadcasted_iota(jnp.int32, (TS, TS), 1)
    L = jnp.where(ri > ci, 1.0, 0.0)

    def prefix(onehot, carry):
        parts = []
        for ti in range(S // TS):
            Mt = onehot[ti * TS:(ti + 1) * TS, :]
            Pt = lax.dot_general(L, Mt, (((1,), (0,)), ((), ())),
                                 preferred_element_type=jnp.float32) + carry
            parts.append(jnp.sum(Pt * Mt, axis=1, keepdims=True))
            carry = carry + jnp.sum(Mt, axis=0, keepdims=True)
        return jnp.concatenate(parts, axis=0), carry

    rank0, carry = prefix(onehot0, jnp.zeros((1, LANES), jnp.float32))
    rank1, cnt = prefix(onehot1, carry)

    nbf = jnp.floor((cnt + (T - 1)) * (1.0 / T))
    r128 = lax.broadcasted_iota(jnp.int32, (LANES, LANES), 0)
    c128 = lax.broadcasted_iota(jnp.int32, (LANES, LANES), 1)
    eye = r128 == c128
    nbcol = jnp.sum(jnp.where(eye, jnp.broadcast_to(nbf, (LANES, LANES)),
                              0.0), axis=1, keepdims=True)
    cum_excl = jnp.sum(jnp.where(r128 < c128,
                                 jnp.broadcast_to(nbcol, (LANES, LANES)),
                                 0.0), axis=0, keepdims=True)
    cum_incl = cum_excl + nbf
    off_row = cum_excl * float(T)
    off_b = jnp.broadcast_to(off_row, (S, LANES))
    off0 = jnp.sum(jnp.where(sel1, off_b, 0.0), axis=1, keepdims=True)
    off1 = jnp.sum(jnp.where(lane == i2, off_b, 0.0), axis=1, keepdims=True)
    dest0 = (off0 + rank0).astype(jnp.int32)
    dest1 = (off1 + rank1).astype(jnp.int32)

    colinc = jnp.sum(jnp.where(eye, jnp.broadcast_to(cum_incl,
                                                     (LANES, LANES)), 0.0),
                     axis=1, keepdims=True)
    cmp = (c128.astype(jnp.float32) >= colinc) & (r128 < NE)
    bexp = jnp.minimum(jnp.sum(cmp.astype(jnp.int32), axis=0, keepdims=True),
                       NE - 1)
    nblk = jnp.sum(jnp.where((r128 < NE) & (c128 == 0),
                             jnp.broadcast_to(nbcol, (LANES, LANES)), 0.0),
                   axis=0, keepdims=True).astype(jnp.int32)

    logits_ref[...] = logits
    dest_ref[...] = (jnp.where(lane == 0, dest0, 0)
                     + jnp.where(lane == 1, dest1, 0))
    w_ref[...] = jnp.where(lane == 0, w0, 0.0) + jnp.where(lane == 1, w1, 0.0)
    row8 = lax.broadcasted_iota(jnp.int32, (8, LANES), 0)
    bexp_ref[...] = jnp.where(row8 == 0, jnp.broadcast_to(bexp, (8, LANES)),
                              jnp.broadcast_to(nblk, (8, LANES)))


def _route(x2d, router_w, router_b):
    rw_pad = jnp.zeros((D, LANES), jnp.float32).at[:, :NE].set(router_w)
    rb_pad = jnp.zeros((1, LANES), jnp.float32).at[:, :NE].set(router_b)
    return pl.pallas_call(
        _route_body,
        out_shape=(
            jax.ShapeDtypeStruct((S, LANES), jnp.float32),
            jax.ShapeDtypeStruct((S, LANES), jnp.int32),
            jax.ShapeDtypeStruct((S, LANES), jnp.float32),
            jax.ShapeDtypeStruct((8, LANES), jnp.int32),
        ),
    )(x2d, rw_pad, rb_pad)


def _ffn_body(scal_ref, xg_ref, w1_ref, b1_ref, w2_ref, b2_ref, g_ref,
              bb_ref, ws_ref, yg_ref):
    @pl.when(pl.program_id(0) < scal_ref[NB])
    def _live():
        xb = xg_ref[...]
        xt = xb.astype(jnp.float32)
        h = lax.dot_general(xb, w1_ref[0], (((1,), (0,)), ((), ())),
                            preferred_element_type=jnp.float32) + b1_ref[0]
        h = 0.5 * h * (1.0 + lax.erf(h * 0.7071067811865476))
        y = lax.dot_general(h.astype(jnp.bfloat16), w2_ref[0],
                            (((1,), (0,)), ((), ())),
                            preferred_element_type=jnp.float32) + b2_ref[0]
        z = y + xt
        mu = jnp.mean(z, axis=-1, keepdims=True)
        zc = z - mu
        var = jnp.mean(zc * zc, axis=-1, keepdims=True)
        ln = zc * lax.rsqrt(var + 1e-5) * g_ref[0] + bb_ref[0]
        rT = lax.broadcasted_iota(jnp.int32, (T, T), 0)
        cT = lax.broadcasted_iota(jnp.int32, (T, T), 1)
        wcol = jnp.sum(jnp.where(rT == cT,
                                 jnp.broadcast_to(ws_ref[0], (T, T)), 0.0),
                       axis=1, keepdims=True)
        yg_ref[...] = ln * wcol


def _ffn(bexp, xg, W1, b1, W2, b2, ln_g, ln_b, wsorted):
    grid_spec = pltpu.PrefetchScalarGridSpec(
        num_scalar_prefetch=1,
        grid=(NB,),
        in_specs=[
            pl.BlockSpec((T, D), lambda b, s: (b, 0)),
            pl.BlockSpec((1, D, H), lambda b, s: (s[b], 0, 0)),
            pl.BlockSpec((1, 1, H), lambda b, s: (s[b], 0, 0)),
            pl.BlockSpec((1, H, D), lambda b, s: (s[b], 0, 0)),
            pl.BlockSpec((1, 1, D), lambda b, s: (s[b], 0, 0)),
            pl.BlockSpec((1, 1, D), lambda b, s: (s[b], 0, 0)),
            pl.BlockSpec((1, 1, D), lambda b, s: (s[b], 0, 0)),
            pl.BlockSpec((1, 1, T), lambda b, s: (b, 0, 0)),
        ],
        out_specs=pl.BlockSpec((T, D), lambda b, s: (b, 0)),
    )
    return pl.pallas_call(
        _ffn_body,
        grid_spec=grid_spec,
        out_shape=jax.ShapeDtypeStruct((NBT, D), jnp.float32),
    )(bexp, xg, W1.astype(jnp.bfloat16), b1.reshape(NE, 1, H),
      W2.astype(jnp.bfloat16), b2.reshape(NE, 1, D),
      ln_g.reshape(NE, 1, D), ln_b.reshape(NE, 1, D),
      wsorted.reshape(NB, 1, T))


@functools.lru_cache(maxsize=None)
def _sc_mesh():
    return plsc.VectorSubcoreMesh(core_axis_name="c", subcore_axis_name="s")


def _dispatch_body(dest2_hbm, w2_hbm, x_hbm, xg_hbm, ws_hbm,
                   dest_v, w_v, tok_v, wsv, buf0, buf1,
                   gs0, gs1, ws0, ws1, wssem):
    wid = lax.axis_index("s") * NC + lax.axis_index("c")
    pltpu.sync_copy(dest2_hbm, dest_v)
    pltpu.sync_copy(w2_hbm, w_v)

    def initbody(i, c):
        for u in range(4):
            tok_v[pl.ds(i * 64 + u * 16, 16)] = jnp.zeros((16,), jnp.int32)
            wsv[pl.ds(i * 64 + u * 16, 16)] = jnp.zeros((16,), jnp.float32)
        return c

    lax.fori_loop(0, NBT // 64, initbody, 0)

    def scatbody(i, c):
        for u in range(4):
            sl = pl.ds(i * 64 + u * 16, 16)
            idx = dest_v[sl]
            pvec = lax.iota(jnp.int32, 16) + (i * 64 + u * 16)
            tvec = pvec - jnp.where(pvec >= S, S, 0)
            plsc.store_scatter(tok_v, [idx], tvec)
            plsc.store_scatter(wsv, [idx], w_v[sl])
        return c

    lax.fori_loop(0, (2 * S) // 64, scatbody, 0)

    base = wid * RPW
    wcp = pltpu.async_copy(wsv.at[pl.ds(base, RPW)],
                           ws_hbm.at[pl.ds(base, RPW)], wssem)
    bufs = [buf0, buf1]
    gsems = [gs0, gs1]
    wsems = [ws0, ws1]
    nrnd = RPW // GCH
    gathers = [None, None]
    writes = [None, None]
    gathers[0] = pltpu.async_copy(
        x_hbm.at[tok_v.at[pl.ds(base, GCH)]], bufs[0], gsems[0])
    for i in range(nrnd):
        b = i % 2
        if i + 1 < nrnd:
            b2 = (i + 1) % 2
            if writes[b2] is not None:
                writes[b2].wait()
            gathers[b2] = pltpu.async_copy(
                x_hbm.at[tok_v.at[pl.ds(base + (i + 1) * GCH, GCH)]],
                bufs[b2], gsems[b2])
        gathers[b].wait()
        writes[b] = pltpu.async_copy(
            bufs[b], xg_hbm.at[pl.ds(base + i * GCH, GCH)], wsems[b])
    writes[(nrnd - 2) % 2].wait()
    writes[(nrnd - 1) % 2].wait()
    wcp.wait()


def _dispatch_sc(dest2, w2, x2d):
    fn = pl.kernel(
        _dispatch_body,
        mesh=_sc_mesh(),
        out_type=[
            jax.ShapeDtypeStruct((NBT, D // 2), jnp.int32),
            jax.ShapeDtypeStruct((NBT,), jnp.float32),
        ],
        scratch_types=[
            pltpu.VMEM((2 * S,), jnp.int32),
            pltpu.VMEM((2 * S,), jnp.float32),
            pltpu.VMEM((NBT,), jnp.int32),
            pltpu.VMEM((NBT,), jnp.float32),
            pltpu.VMEM((GCH, D // 2), jnp.int32),
            pltpu.VMEM((GCH, D // 2), jnp.int32),
            pltpu.SemaphoreType.DMA,
            pltpu.SemaphoreType.DMA,
            pltpu.SemaphoreType.DMA,
            pltpu.SemaphoreType.DMA,
            pltpu.SemaphoreType.DMA,
        ],
        compiler_params=pltpu.CompilerParams(needs_layout_passes=False),
    )
    return fn(dest2, w2, x2d)


def _combine_body(dest2_hbm, yg_hbm, out_hbm, d0_v, d1_v,
                  a0, a1, c0, c1, g0a, g1a, wa, g0b, g1b, wb):
    wid = lax.axis_index("s") * NC + lax.axis_index("c")
    tbase = wid * TPW
    pltpu.sync_copy(dest2_hbm.at[pl.ds(tbase, TPW)], d0_v)
    pltpu.sync_copy(dest2_hbm.at[pl.ds(S + tbase, TPW)], d1_v)
    nrnd = TPW // CCH
    sets = [(a0, a1, g0a, g1a, wa), (c0, c1, g0b, g1b, wb)]
    gath = [None, None]
    writes = [None, None]

    def start_gathers(rnd):
        b = rnd % 2
        s0, s1, gs0, gs1, _ = sets[b]
        gath[b] = (
            pltpu.async_copy(yg_hbm.at[d0_v.at[pl.ds(rnd * CCH, CCH)]],
                             s0, gs0),
            pltpu.async_copy(yg_hbm.at[d1_v.at[pl.ds(rnd * CCH, CCH)]],
                             s1, gs1),
        )

    start_gathers(0)
    for i in range(nrnd):
        b = i % 2
        s0, s1, _, _, wsem = sets[b]
        if i + 1 < nrnd:
            b2 = (i + 1) % 2
            if writes[b2] is not None:
                writes[b2].wait()
            start_gathers(i + 1)
        gath[b][0].wait()
        gath[b][1].wait()

        def rowbody(r, c, s0=s0, s1=s1):
            for u in range(D // 16):
                sl = pl.ds(u * 16, 16)
                s0[r, sl] = s0[r, sl] + s1[r, sl]
            return c

        lax.fori_loop(0, CCH, rowbody, 0)
        writes[b] = pltpu.async_copy(
            s0, out_hbm.at[pl.ds(tbase + i * CCH, CCH)], wsem)
    writes[(nrnd - 2) % 2].wait()
    writes[(nrnd - 1) % 2].wait()


def _combine_sc(dest2, yg):
    fn = pl.kernel(
        _combine_body,
        mesh=_sc_mesh(),
        out_type=jax.ShapeDtypeStruct((S, D), jnp.float32),
        scratch_types=[
            pltpu.VMEM((TPW,), jnp.int32),
            pltpu.VMEM((TPW,), jnp.int32),
            pltpu.VMEM((CCH, D), jnp.float32),
            pltpu.VMEM((CCH, D), jnp.float32),
            pltpu.VMEM((CCH, D), jnp.float32),
            pltpu.VMEM((CCH, D), jnp.float32),
            pltpu.SemaphoreType.DMA,
            pltpu.SemaphoreType.DMA,
            pltpu.SemaphoreType.DMA,
            pltpu.SemaphoreType.DMA,
            pltpu.SemaphoreType.DMA,
            pltpu.SemaphoreType.DMA,
        ],
        compiler_params=pltpu.CompilerParams(needs_layout_passes=False),
    )
    return fn(dest2, yg)


@jax.jit
def kernel(x, router_w, router_b, W1, b1, W2, b2, ln_g, ln_b):
    x2d = x.reshape(S, D)
    logits_pad, dest_pad, w_pad, bexp_pad = _route(x2d, router_w, router_b)
    dest2 = dest_pad[:, :2].T.reshape(2 * S)
    w2 = w_pad[:, :2].T.reshape(2 * S)
    bexp = jnp.concatenate([bexp_pad[0, :NB], bexp_pad[1, :1]])
    xi = lax.bitcast_convert_type(
        x2d.astype(jnp.bfloat16).reshape(S, D // 2, 2), jnp.int32)
    xgi, wsorted = _dispatch_sc(dest2, w2, xi)
    xg = lax.bitcast_convert_type(xgi, jnp.bfloat16).reshape(NBT, D)
    yg = _ffn(bexp, xg, W1, b1, W2, b2, ln_g, ln_b, wsorted)
    out = _combine_sc(dest2, yg)
    return (out.reshape(1, S, D), logits_pad[:, :NE].reshape(1, S, NE))


# speedup vs baseline: 1.8679x; 1.8679x over previous
import functools
import jax
import jax.numpy as jnp
from jax import lax
from jax.experimental import pallas as pl
from jax.experimental.pallas import tpu as pltpu
from jax.experimental.pallas import tpu_sc as plsc

NE = 8
D = 1024
H = 2048
S = 2048
LANES = 128
T = 256
NB = 24
NBT = NB * T
TS = 512

NC = 2
NS = 16
NW = NC * NS
RPW = NBT // NW
GCH = 48
TPW = S // NW
CCH = 16


def _route_body(x_ref, rw_ref, rb_ref, logits_ref, dest_ref, w_ref,
                bexp_ref, xi_ref):
    x = x_ref[...]
    u_lo = lax.bitcast_convert_type(x[:, :D // 2], jnp.uint32)
    u_hi = lax.bitcast_convert_type(x[:, D // 2:], jnp.uint32)
    r_lo = u_lo + jnp.uint32(0x7FFF) + ((u_lo >> 16) & jnp.uint32(1))
    r_hi = u_hi + jnp.uint32(0x7FFF) + ((u_hi >> 16) & jnp.uint32(1))
    packed = (r_hi & jnp.uint32(0xFFFF0000)) | (r_lo >> 16)
    xi_ref[...] = lax.bitcast_convert_type(packed, jnp.int32)
    logits = lax.dot_general(
        x, rw_ref[...], (((1,), (0,)), ((), ())),
        preferred_element_type=jnp.float32) + rb_ref[...]
    lane = lax.broadcasted_iota(jnp.int32, (S, LANES), 1)
    valid = lane < NE
    lmask = jnp.where(valid, logits, -jnp.inf)
    mx = jnp.max(lmask, axis=-1, keepdims=True)
    p = jnp.exp(lmask - mx)
    probs = p / jnp.sum(p, axis=-1, keepdims=True)
    m1 = jnp.max(probs, axis=-1, keepdims=True)
    i1 = jnp.min(jnp.where((probs == m1) & valid, lane, LANES), axis=-1,
                 keepdims=True)
    sel1 = lane == i1
    probs2 = jnp.where(sel1, -1.0, probs)
    m2 = jnp.max(probs2, axis=-1, keepdims=True)
    i2 = jnp.min(jnp.where((probs2 == m2) & valid, lane, LANES), axis=-1,
                 keepdims=True)
    wsum = m1 + m2
    w0 = m1 / wsum
    w1 = m2 / wsum

    onehot0 = jnp.where(sel1, 1.0, 0.0)
    onehot1 = jnp.where(lane == i2, 1.0, 0.0)

    ri = lax.broadcasted_iota(jnp.int32, (TS, TS), 0)
    ci = lax.broadcasted_iota(jnp.int32, (TS, TS), 1)
    L = jnp.where(ri > ci, 1.0, 0.0)

    def prefix(onehot, carry):
        parts = []
        for ti in range(S // TS):
            Mt = onehot[ti * TS:(ti + 1) * TS, :]
            Pt = lax.dot_general(L, Mt, (((1,), (0,)), ((), ())),
                                 preferred_element_type=jnp.float32) + carry
            parts.append(jnp.sum(Pt * Mt, axis=1, keepdims=True))
            carry = carry + jnp.sum(Mt, axis=0, keepdims=True)
        return jnp.concatenate(parts, axis=0), carry

    rank0, carry = prefix(onehot0, jnp.zeros((1, LANES), jnp.float32))
    rank1, cnt = prefix(onehot1, carry)

    nbf = jnp.floor((cnt + (T - 1)) * (1.0 / T))
    r128 = lax.broadcasted_iota(jnp.int32, (LANES, LANES), 0)
    c128 = lax.broadcasted_iota(jnp.int32, (LANES, LANES), 1)
    eye = r128 == c128
    nbcol = jnp.sum(jnp.where(eye, jnp.broadcast_to(nbf, (LANES, LANES)),
                              0.0), axis=1, keepdims=True)
    cum_excl = jnp.sum(jnp.where(r128 < c128,
                                 jnp.broadcast_to(nbcol, (LANES, LANES)),
                                 0.0), axis=0, keepdims=True)
    cum_incl = cum_excl + nbf
    off_row = cum_excl * float(T)
    off_b = jnp.broadcast_to(off_row, (S, LANES))
    off0 = jnp.sum(jnp.where(sel1, off_b, 0.0), axis=1, keepdims=True)
    off1 = jnp.sum(jnp.where(lane == i2, off_b, 0.0), axis=1, keepdims=True)
    dest0 = (off0 + rank0).astype(jnp.int32)
    dest1 = (off1 + rank1).astype(jnp.int32)

    colinc = jnp.sum(jnp.where(eye, jnp.broadcast_to(cum_incl,
                                                     (LANES, LANES)), 0.0),
                     axis=1, keepdims=True)
    cmp = (c128.astype(jnp.float32) >= colinc) & (r128 < NE)
    bexp = jnp.minimum(jnp.sum(cmp.astype(jnp.int32), axis=0, keepdims=True),
                       NE - 1)
    nblk = jnp.sum(jnp.where((r128 < NE) & (c128 == 0),
                             jnp.broadcast_to(nbcol, (LANES, LANES)), 0.0),
                   axis=0, keepdims=True).astype(jnp.int32)

    logits_ref[...] = logits
    dest_ref[...] = (jnp.where(lane == 0, dest0, 0)
                     + jnp.where(lane == 1, dest1, 0))
    w_ref[...] = jnp.where(lane == 0, w0, 0.0) + jnp.where(lane == 1, w1, 0.0)
    row8 = lax.broadcasted_iota(jnp.int32, (8, LANES), 0)
    bexp_ref[...] = jnp.where(row8 == 0, jnp.broadcast_to(bexp, (8, LANES)),
                              jnp.broadcast_to(nblk, (8, LANES)))


def _route(x2d, router_w, router_b):
    rw_pad = jnp.zeros((D, LANES), jnp.float32).at[:, :NE].set(router_w)
    rb_pad = jnp.zeros((1, LANES), jnp.float32).at[:, :NE].set(router_b)
    return pl.pallas_call(
        _route_body,
        out_shape=(
            jax.ShapeDtypeStruct((S, LANES), jnp.float32),
            jax.ShapeDtypeStruct((S, LANES), jnp.int32),
            jax.ShapeDtypeStruct((S, LANES), jnp.float32),
            jax.ShapeDtypeStruct((8, LANES), jnp.int32),
            jax.ShapeDtypeStruct((S, D // 2), jnp.int32),
        ),
    )(x2d, rw_pad, rb_pad)


def _ffn_body(scal_ref, xg_ref, w1_ref, b1_ref, w2_ref, b2_ref, g_ref,
              bb_ref, ws_ref, yg_ref):
    @pl.when(pl.program_id(0) < scal_ref[NB])
    def _live():
        u = lax.bitcast_convert_type(xg_ref[...], jnp.uint32)
        f_lo = lax.bitcast_convert_type(u << 16, jnp.float32)
        f_hi = lax.bitcast_convert_type(u & jnp.uint32(0xFFFF0000),
                                        jnp.float32)
        xt = jnp.concatenate([f_lo, f_hi], axis=1)
        h = lax.dot_general(xt, w1_ref[0], (((1,), (0,)), ((), ())),
                            preferred_element_type=jnp.float32) + b1_ref[0]
        h = 0.5 * h * (1.0 + lax.erf(h * 0.7071067811865476))
        y = lax.dot_general(h, w2_ref[0], (((1,), (0,)), ((), ())),
                            preferred_element_type=jnp.float32) + b2_ref[0]
        z = y + xt
        mu = jnp.mean(z, axis=-1, keepdims=True)
        zc = z - mu
        var = jnp.mean(zc * zc, axis=-1, keepdims=True)
        ln = zc * lax.rsqrt(var + 1e-5) * g_ref[0] + bb_ref[0]
        rT = lax.broadcasted_iota(jnp.int32, (T, T), 0)
        cT = lax.broadcasted_iota(jnp.int32, (T, T), 1)
        wcol = jnp.sum(jnp.where(rT == cT,
                                 jnp.broadcast_to(ws_ref[0], (T, T)), 0.0),
                       axis=1, keepdims=True)
        yg_ref[...] = ln * wcol


def _ffn(bexp, xg, W1, b1, W2, b2, ln_g, ln_b, wsorted):
    grid_spec = pltpu.PrefetchScalarGridSpec(
        num_scalar_prefetch=1,
        grid=(NB,),
        in_specs=[
            pl.BlockSpec((T, D // 2), lambda b, s: (b, 0)),
            pl.BlockSpec((1, D, H), lambda b, s: (s[b], 0, 0)),
            pl.BlockSpec((1, 1, H), lambda b, s: (s[b], 0, 0)),
            pl.BlockSpec((1, H, D), lambda b, s: (s[b], 0, 0)),
            pl.BlockSpec((1, 1, D), lambda b, s: (s[b], 0, 0)),
            pl.BlockSpec((1, 1, D), lambda b, s: (s[b], 0, 0)),
            pl.BlockSpec((1, 1, D), lambda b, s: (s[b], 0, 0)),
            pl.BlockSpec((1, 1, T), lambda b, s: (b, 0, 0)),
        ],
        out_specs=pl.BlockSpec((T, D), lambda b, s: (b, 0)),
    )
    return pl.pallas_call(
        _ffn_body,
        grid_spec=grid_spec,
        out_shape=jax.ShapeDtypeStruct((NBT, D), jnp.float32),
    )(bexp, xg, W1, b1.reshape(NE, 1, H),
      W2, b2.reshape(NE, 1, D),
      ln_g.reshape(NE, 1, D), ln_b.reshape(NE, 1, D),
      wsorted.reshape(NB, 1, T))


@functools.lru_cache(maxsize=None)
def _sc_mesh():
    return plsc.VectorSubcoreMesh(core_axis_name="c", subcore_axis_name="s")


def _dispatch_body(dest2_hbm, w2_hbm, x_hbm, xg_hbm, ws_hbm,
                   dest_v, w_v, tok_v, wsv, buf0, buf1,
                   gs0, gs1, ws0, ws1, wssem):
    wid = lax.axis_index("s") * NC + lax.axis_index("c")
    pltpu.sync_copy(dest2_hbm, dest_v)
    pltpu.sync_copy(w2_hbm, w_v)

    def initbody(i, c):
        for u in range(4):
            tok_v[pl.ds(i * 64 + u * 16, 16)] = jnp.zeros((16,), jnp.int32)
            wsv[pl.ds(i * 64 + u * 16, 16)] = jnp.zeros((16,), jnp.float32)
        return c

    lax.fori_loop(0, NBT // 64, initbody, 0)

    def scatbody(i, c):
        for u in range(4):
            sl = pl.ds(i * 64 + u * 16, 16)
            idx = dest_v[sl]
            pvec = lax.iota(jnp.int32, 16) + (i * 64 + u * 16)
            tvec = pvec - jnp.where(pvec >= S, S, 0)
            plsc.store_scatter(tok_v, [idx], tvec)
            plsc.store_scatter(wsv, [idx], w_v[sl])
        return c

    lax.fori_loop(0, (2 * S) // 64, scatbody, 0)

    base = wid * RPW
    wcp = pltpu.async_copy(wsv.at[pl.ds(base, RPW)],
                           ws_hbm.at[pl.ds(base, RPW)], wssem)
    bufs = [buf0, buf1]
    gsems = [gs0, gs1]
    wsems = [ws0, ws1]
    nrnd = RPW // GCH
    gathers = [None, None]
    writes = [None, None]
    gathers[0] = pltpu.async_copy(
        x_hbm.at[tok_v.at[pl.ds(base, GCH)]], bufs[0], gsems[0])
    for i in range(nrnd):
        b = i % 2
        if i + 1 < nrnd:
            b2 = (i + 1) % 2
            if writes[b2] is not None:
                writes[b2].wait()
            gathers[b2] = pltpu.async_copy(
                x_hbm.at[tok_v.at[pl.ds(base + (i + 1) * GCH, GCH)]],
                bufs[b2], gsems[b2])
        gathers[b].wait()
        writes[b] = pltpu.async_copy(
            bufs[b], xg_hbm.at[pl.ds(base + i * GCH, GCH)], wsems[b])
    writes[(nrnd - 2) % 2].wait()
    writes[(nrnd - 1) % 2].wait()
    wcp.wait()


def _dispatch_sc(dest2, w2, x2d):
    fn = pl.kernel(
        _dispatch_body,
        mesh=_sc_mesh(),
        out_type=[
            jax.ShapeDtypeStruct((NBT, D // 2), jnp.int32),
            jax.ShapeDtypeStruct((NBT,), jnp.float32),
        ],
        scratch_types=[
            pltpu.VMEM((2 * S,), jnp.int32),
            pltpu.VMEM((2 * S,), jnp.float32),
            pltpu.VMEM((NBT,), jnp.int32),
            pltpu.VMEM((NBT,), jnp.float32),
            pltpu.VMEM((GCH, D // 2), jnp.int32),
            pltpu.VMEM((GCH, D // 2), jnp.int32),
            pltpu.SemaphoreType.DMA,
            pltpu.SemaphoreType.DMA,
            pltpu.SemaphoreType.DMA,
            pltpu.SemaphoreType.DMA,
            pltpu.SemaphoreType.DMA,
        ],
        compiler_params=pltpu.CompilerParams(needs_layout_passes=False),
    )
    return fn(dest2, w2, x2d)


def _combine_body(dest2_hbm, yg_hbm, out_hbm, d0_v, d1_v,
                  a0, a1, c0, c1, g0a, g1a, wa, g0b, g1b, wb):
    wid = lax.axis_index("s") * NC + lax.axis_index("c")
    tbase = wid * TPW
    pltpu.sync_copy(dest2_hbm.at[pl.ds(tbase, TPW)], d0_v)
    pltpu.sync_copy(dest2_hbm.at[pl.ds(S + tbase, TPW)], d1_v)
    nrnd = TPW // CCH
    sets = [(a0, a1, g0a, g1a, wa), (c0, c1, g0b, g1b, wb)]
    gath = [None, None]
    writes = [None, None]

    def start_gathers(rnd):
        b = rnd % 2
        s0, s1, gs0, gs1, _ = sets[b]
        gath[b] = (
            pltpu.async_copy(yg_hbm.at[d0_v.at[pl.ds(rnd * CCH, CCH)]],
                             s0, gs0),
            pltpu.async_copy(yg_hbm.at[d1_v.at[pl.ds(rnd * CCH, CCH)]],
                             s1, gs1),
        )

    start_gathers(0)
    for i in range(nrnd):
        b = i % 2
        s0, s1, _, _, wsem = sets[b]
        if i + 1 < nrnd:
            b2 = (i + 1) % 2
            if writes[b2] is not None:
                writes[b2].wait()
            start_gathers(i + 1)
        gath[b][0].wait()
        gath[b][1].wait()

        def rowbody(r, c, s0=s0, s1=s1):
            for u in range(D // 16):
                sl = pl.ds(u * 16, 16)
                s0[r, sl] = s0[r, sl] + s1[r, sl]
            return c

        lax.fori_loop(0, CCH, rowbody, 0)
        writes[b] = pltpu.async_copy(
            s0, out_hbm.at[pl.ds(tbase + i * CCH, CCH)], wsem)
    writes[(nrnd - 2) % 2].wait()
    writes[(nrnd - 1) % 2].wait()


def _combine_sc(dest2, yg):
    fn = pl.kernel(
        _combine_body,
        mesh=_sc_mesh(),
        out_type=jax.ShapeDtypeStruct((S, D), jnp.float32),
        scratch_types=[
            pltpu.VMEM((TPW,), jnp.int32),
            pltpu.VMEM((TPW,), jnp.int32),
            pltpu.VMEM((CCH, D), jnp.float32),
            pltpu.VMEM((CCH, D), jnp.float32),
            pltpu.VMEM((CCH, D), jnp.float32),
            pltpu.VMEM((CCH, D), jnp.float32),
            pltpu.SemaphoreType.DMA,
            pltpu.SemaphoreType.DMA,
            pltpu.SemaphoreType.DMA,
            pltpu.SemaphoreType.DMA,
            pltpu.SemaphoreType.DMA,
            pltpu.SemaphoreType.DMA,
        ],
        compiler_params=pltpu.CompilerParams(needs_layout_passes=False),
    )
    return fn(dest2, yg)


@jax.jit
def kernel(x, router_w, router_b, W1, b1, W2, b2, ln_g, ln_b):
    x2d = x.reshape(S, D)
    logits_pad, dest_pad, w_pad, bexp_pad, xi = _route(x2d, router_w,
                                                       router_b)
    dest2 = dest_pad[:, :2].T.reshape(2 * S)
    w2 = w_pad[:, :2].T.reshape(2 * S)
    bexp = jnp.concatenate([bexp_pad[0, :NB], bexp_pad[1, :1]])
    xgi, wsorted = _dispatch_sc(dest2, w2, xi)
    yg = _ffn(bexp, xgi, W1, b1, W2, b2, ln_g, ln_b, wsorted)
    out = _combine_sc(dest2, yg)
    return (out.reshape(1, S, D), logits_pad[:, :NE].reshape(1, S, NE))


# speedup vs baseline: 2.7641x; 1.4798x over previous
import functools
import jax
import jax.numpy as jnp
from jax import lax
from jax.experimental import pallas as pl
from jax.experimental.pallas import tpu as pltpu
from jax.experimental.pallas import tpu_sc as plsc

NE = 8
D = 1024
H = 2048
S = 2048
LANES = 128
T = 256
NB = 24
NBT = NB * T
TS = 512

NC = 2
NS = 16
NW = NC * NS
RPW = NBT // NW
GCH = 48
TPW = S // NW
CCH = 16


def _route_body(x_ref, rw_ref, rb_ref, logits_ref, dest_ref, w_ref,
                bexp_ref, xb_ref):
    x = x_ref[...]
    xb_ref[...] = x.astype(jnp.bfloat16)
    logits = lax.dot_general(
        x, rw_ref[...], (((1,), (0,)), ((), ())),
        preferred_element_type=jnp.float32) + rb_ref[...]
    lane = lax.broadcasted_iota(jnp.int32, (S, LANES), 1)
    valid = lane < NE
    lmask = jnp.where(valid, logits, -jnp.inf)
    mx = jnp.max(lmask, axis=-1, keepdims=True)
    p = jnp.exp(lmask - mx)
    probs = p / jnp.sum(p, axis=-1, keepdims=True)
    m1 = jnp.max(probs, axis=-1, keepdims=True)
    i1 = jnp.min(jnp.where((probs == m1) & valid, lane, LANES), axis=-1,
                 keepdims=True)
    sel1 = lane == i1
    probs2 = jnp.where(sel1, -1.0, probs)
    m2 = jnp.max(probs2, axis=-1, keepdims=True)
    i2 = jnp.min(jnp.where((probs2 == m2) & valid, lane, LANES), axis=-1,
                 keepdims=True)
    wsum = m1 + m2
    w0 = m1 / wsum
    w1 = m2 / wsum

    onehot0 = jnp.where(sel1, 1.0, 0.0)
    onehot1 = jnp.where(lane == i2, 1.0, 0.0)

    ri = lax.broadcasted_iota(jnp.int32, (TS, TS), 0)
    ci = lax.broadcasted_iota(jnp.int32, (TS, TS), 1)
    L = jnp.where(ri > ci, 1.0, 0.0)

    def prefix(onehot, carry):
        parts = []
        for ti in range(S // TS):
            Mt = onehot[ti * TS:(ti + 1) * TS, :]
            Pt = lax.dot_general(L, Mt, (((1,), (0,)), ((), ())),
                                 preferred_element_type=jnp.float32) + carry
            parts.append(jnp.sum(Pt * Mt, axis=1, keepdims=True))
            carry = carry + jnp.sum(Mt, axis=0, keepdims=True)
        return jnp.concatenate(parts, axis=0), carry

    rank0, carry = prefix(onehot0, jnp.zeros((1, LANES), jnp.float32))
    rank1, cnt = prefix(onehot1, carry)

    nbf = jnp.floor((cnt + (T - 1)) * (1.0 / T))
    r128 = lax.broadcasted_iota(jnp.int32, (LANES, LANES), 0)
    c128 = lax.broadcasted_iota(jnp.int32, (LANES, LANES), 1)
    eye = r128 == c128
    nbcol = jnp.sum(jnp.where(eye, jnp.broadcast_to(nbf, (LANES, LANES)),
                              0.0), axis=1, keepdims=True)
    cum_excl = jnp.sum(jnp.where(r128 < c128,
                                 jnp.broadcast_to(nbcol, (LANES, LANES)),
                                 0.0), axis=0, keepdims=True)
    cum_incl = cum_excl + nbf
    off_row = cum_excl * float(T)
    off_b = jnp.broadcast_to(off_row, (S, LANES))
    off0 = jnp.sum(jnp.where(sel1, off_b, 0.0), axis=1, keepdims=True)
    off1 = jnp.sum(jnp.where(lane == i2, off_b, 0.0), axis=1, keepdims=True)
    dest0 = (off0 + rank0).astype(jnp.int32)
    dest1 = (off1 + rank1).astype(jnp.int32)

    colinc = jnp.sum(jnp.where(eye, jnp.broadcast_to(cum_incl,
                                                     (LANES, LANES)), 0.0),
                     axis=1, keepdims=True)
    cmp = (c128.astype(jnp.float32) >= colinc) & (r128 < NE)
    bexp = jnp.minimum(jnp.sum(cmp.astype(jnp.int32), axis=0, keepdims=True),
                       NE - 1)
    nblk = jnp.sum(jnp.where((r128 < NE) & (c128 == 0),
                             jnp.broadcast_to(nbcol, (LANES, LANES)), 0.0),
                   axis=0, keepdims=True).astype(jnp.int32)

    logits_ref[...] = logits
    dest_ref[...] = (jnp.where(lane == 0, dest0, 0)
                     + jnp.where(lane == 1, dest1, 0))
    w_ref[...] = jnp.where(lane == 0, w0, 0.0) + jnp.where(lane == 1, w1, 0.0)
    row8 = lax.broadcasted_iota(jnp.int32, (8, LANES), 0)
    bexp_ref[...] = jnp.where(row8 == 0, jnp.broadcast_to(bexp, (8, LANES)),
                              jnp.broadcast_to(nblk, (8, LANES)))


def _route(x2d, router_w, router_b):
    rw_pad = jnp.zeros((D, LANES), jnp.float32).at[:, :NE].set(router_w)
    rb_pad = jnp.zeros((1, LANES), jnp.float32).at[:, :NE].set(router_b)
    return pl.pallas_call(
        _route_body,
        out_shape=(
            jax.ShapeDtypeStruct((S, LANES), jnp.float32),
            jax.ShapeDtypeStruct((S, LANES), jnp.int32),
            jax.ShapeDtypeStruct((S, LANES), jnp.float32),
            jax.ShapeDtypeStruct((8, LANES), jnp.int32),
            jax.ShapeDtypeStruct((S, D), jnp.bfloat16),
        ),
    )(x2d, rw_pad, rb_pad)


def _ffn_body(scal_ref, d2_ref, w2_ref_in, xb_ref, w1_ref, b1_ref, w2_ref,
              b2_ref, g_ref, bb_ref, yg_ref):
    b = pl.program_id(0)

    @pl.when(b < scal_ref[NB])
    def _live():
        dd = d2_ref[...]
        ww = w2_ref_in[...]
        ri = lax.broadcasted_iota(jnp.int32, (T, S), 0) + b * T
        p0 = jnp.broadcast_to(dd[0:1, :], (T, S)) == ri
        p1 = jnp.broadcast_to(dd[1:2, :], (T, S)) == ri
        pb = (p0 | p1).astype(jnp.bfloat16)
        wcol = jnp.sum(jnp.where(p0, jnp.broadcast_to(ww[0:1, :], (T, S)),
                                 0.0)
                       + jnp.where(p1, jnp.broadcast_to(ww[1:2, :], (T, S)),
                                   0.0), axis=1, keepdims=True)
        xt = lax.dot_general(pb, xb_ref[...], (((1,), (0,)), ((), ())),
                             preferred_element_type=jnp.float32)
        h = lax.dot_general(xt, w1_ref[0], (((1,), (0,)), ((), ())),
                            preferred_element_type=jnp.float32) + b1_ref[0]
        h = 0.5 * h * (1.0 + lax.erf(h * 0.7071067811865476))
        y = lax.dot_general(h, w2_ref[0], (((1,), (0,)), ((), ())),
                            preferred_element_type=jnp.float32) + b2_ref[0]
        z = y + xt
        mu = jnp.mean(z, axis=-1, keepdims=True)
        zc = z - mu
        var = jnp.mean(zc * zc, axis=-1, keepdims=True)
        ln = zc * lax.rsqrt(var + 1e-5) * g_ref[0] + bb_ref[0]
        yg_ref[...] = ln * wcol


def _ffn(bexp, dest2, w2, xb, W1, b1, W2, b2, ln_g, ln_b):
    grid_spec = pltpu.PrefetchScalarGridSpec(
        num_scalar_prefetch=1,
        grid=(NB,),
        in_specs=[
            pl.BlockSpec((2, S), lambda b, s: (0, 0)),
            pl.BlockSpec((2, S), lambda b, s: (0, 0)),
            pl.BlockSpec((S, D), lambda b, s: (0, 0)),
            pl.BlockSpec((1, D, H), lambda b, s: (s[b], 0, 0)),
            pl.BlockSpec((1, 1, H), lambda b, s: (s[b], 0, 0)),
            pl.BlockSpec((1, H, D), lambda b, s: (s[b], 0, 0)),
            pl.BlockSpec((1, 1, D), lambda b, s: (s[b], 0, 0)),
            pl.BlockSpec((1, 1, D), lambda b, s: (s[b], 0, 0)),
            pl.BlockSpec((1, 1, D), lambda b, s: (s[b], 0, 0)),
        ],
        out_specs=pl.BlockSpec((T, D), lambda b, s: (b, 0)),
    )
    return pl.pallas_call(
        _ffn_body,
        grid_spec=grid_spec,
        out_shape=jax.ShapeDtypeStruct((NBT, D), jnp.float32),
    )(bexp, dest2, w2, xb, W1, b1.reshape(NE, 1, H),
      W2, b2.reshape(NE, 1, D),
      ln_g.reshape(NE, 1, D), ln_b.reshape(NE, 1, D))


@functools.lru_cache(maxsize=None)
def _sc_mesh():
    return plsc.VectorSubcoreMesh(core_axis_name="c", subcore_axis_name="s")


def _combine_body(dest2_hbm, yg_hbm, out_hbm, d0_v, d1_v,
                  a0, a1, c0, c1, g0a, g1a, wa, g0b, g1b, wb):
    wid = lax.axis_index("s") * NC + lax.axis_index("c")
    tbase = wid * TPW
    pltpu.sync_copy(dest2_hbm.at[pl.ds(tbase, TPW)], d0_v)
    pltpu.sync_copy(dest2_hbm.at[pl.ds(S + tbase, TPW)], d1_v)
    nrnd = TPW // CCH
    sets = [(a0, a1, g0a, g1a, wa), (c0, c1, g0b, g1b, wb)]
    gath = [None, None]
    writes = [None, None]

    def start_gathers(rnd):
        b = rnd % 2
        s0, s1, gs0, gs1, _ = sets[b]
        gath[b] = (
            pltpu.async_copy(yg_hbm.at[d0_v.at[pl.ds(rnd * CCH, CCH)]],
                             s0, gs0),
            pltpu.async_copy(yg_hbm.at[d1_v.at[pl.ds(rnd * CCH, CCH)]],
                             s1, gs1),
        )

    start_gathers(0)
    for i in range(nrnd):
        b = i % 2
        s0, s1, _, _, wsem = sets[b]
        if i + 1 < nrnd:
            b2 = (i + 1) % 2
            if writes[b2] is not None:
                writes[b2].wait()
            start_gathers(i + 1)
        gath[b][0].wait()
        gath[b][1].wait()

        def rowbody(r, c, s0=s0, s1=s1):
            for u in range(D // 16):
                sl = pl.ds(u * 16, 16)
                s0[r, sl] = s0[r, sl] + s1[r, sl]
            return c

        lax.fori_loop(0, CCH, rowbody, 0)
        writes[b] = pltpu.async_copy(
            s0, out_hbm.at[pl.ds(tbase + i * CCH, CCH)], wsem)
    writes[(nrnd - 2) % 2].wait()
    writes[(nrnd - 1) % 2].wait()


def _combine_sc(dest2, yg):
    fn = pl.kernel(
        _combine_body,
        mesh=_sc_mesh(),
        out_type=jax.ShapeDtypeStruct((S, D), jnp.float32),
        scratch_types=[
            pltpu.VMEM((TPW,), jnp.int32),
            pltpu.VMEM((TPW,), jnp.int32),
            pltpu.VMEM((CCH, D), jnp.float32),
            pltpu.VMEM((CCH, D), jnp.float32),
            pltpu.VMEM((CCH, D), jnp.float32),
            pltpu.VMEM((CCH, D), jnp.float32),
            pltpu.SemaphoreType.DMA,
            pltpu.SemaphoreType.DMA,
            pltpu.SemaphoreType.DMA,
            pltpu.SemaphoreType.DMA,
            pltpu.SemaphoreType.DMA,
            pltpu.SemaphoreType.DMA,
        ],
        compiler_params=pltpu.CompilerParams(needs_layout_passes=False),
    )
    return fn(dest2, yg)


@jax.jit
def kernel(x, router_w, router_b, W1, b1, W2, b2, ln_g, ln_b):
    x2d = x.reshape(S, D)
    logits_pad, dest_pad, w_pad, bexp_pad, xb = _route(x2d, router_w,
                                                       router_b)
    dest2m = dest_pad[:, :2].T
    w2m = w_pad[:, :2].T
    bexp = jnp.concatenate([bexp_pad[0, :NB], bexp_pad[1, :1]])
    yg = _ffn(bexp, dest2m, w2m, xb, W1, b1, W2, b2, ln_g, ln_b)
    out = _combine_sc(dest2m.reshape(2 * S), yg)
    return (out.reshape(1, S, D), logits_pad[:, :NE].reshape(1, S, NE))
